# baseline (speedup 1.0000x reference)
import jax
import jax.numpy as jnp
from jax import lax
from jax.experimental import pallas as pl
from jax.experimental.pallas import tpu as pltpu

ROWS = 16384
D = 2048
CH = 512
NCH = ROWS // CH
NBUF = 6


def _body(x_hbm, wt_ref, o_ref, xbufs, pbuf, sems):
    wt = wt_ref[...].astype(jnp.bfloat16)

    def start_dma(c, slot):
        pltpu.make_async_copy(
            x_hbm.at[pl.ds(c * CH, CH)], xbufs.at[slot], sems.at[slot]
        ).start()

    def wait_dma(c, slot):
        pltpu.make_async_copy(
            x_hbm.at[pl.ds(c * CH, CH)], xbufs.at[slot], sems.at[slot]
        ).wait()

    for c in range(NBUF - 1):
        start_dma(c, c)

    def step(i, _):
        nxt = i + NBUF - 1

        @pl.when(nxt < NCH)
        def _():
            start_dma(nxt, lax.rem(nxt, NBUF))

        slot = lax.rem(i, NBUF)
        wait_dma(i, slot)
        xb = xbufs[slot].astype(jnp.bfloat16)
        pbuf[pl.ds(i * CH, CH), :] = jnp.dot(
            xb, wt, preferred_element_type=jnp.float32)
        return 0

    lax.fori_loop(0, NCH, step, 0)
    p = pbuf[...]
    o_ref[0, :] = jnp.sqrt(jnp.sum(p * p, axis=1))


def kernel(x, routers, expert_idx):
    w = lax.dynamic_index_in_dim(routers, expert_idx, axis=0,
                                 keepdims=False)
    x2 = x.reshape(ROWS, D)
    out = pl.pallas_call(
        _body,
        in_specs=[
            pl.BlockSpec(memory_space=pl.ANY),
            pl.BlockSpec(memory_space=pltpu.VMEM),
        ],
        out_specs=pl.BlockSpec(memory_space=pltpu.VMEM),
        out_shape=jax.ShapeDtypeStruct((1, ROWS), jnp.float32),
        scratch_shapes=[
            pltpu.VMEM((NBUF, CH, D), jnp.float32),
            pltpu.VMEM((ROWS, 8), jnp.float32),
            pltpu.SemaphoreType.DMA((NBUF,)),
        ],
    )(x2, w.T)
    return out.reshape(4, 4096)


# speedup vs baseline: 1.1608x; 1.1608x over previous
import jax
import jax.numpy as jnp
from jax import lax
from jax.experimental import pallas as pl
from jax.experimental.pallas import tpu as pltpu

ROWS = 16384
D = 2048
CH = 512
NCH = ROWS // CH
NBUF = 6


def _body(x_hbm, wt_ref, o_ref, xbufs, sems):
    wt = wt_ref[...].astype(jnp.bfloat16)

    def start_dma(c, slot):
        pltpu.make_async_copy(
            x_hbm.at[pl.ds(c * CH, CH)], xbufs.at[slot], sems.at[slot]
        ).start()

    def wait_dma(c, slot):
        pltpu.make_async_copy(
            x_hbm.at[pl.ds(c * CH, CH)], xbufs.at[slot], sems.at[slot]
        ).wait()

    for c in range(NBUF - 1):
        start_dma(c, c)

    def step(i, _):
        nxt = i + NBUF - 1

        @pl.when(nxt < NCH)
        def _():
            start_dma(nxt, lax.rem(nxt, NBUF))

        slot = lax.rem(i, NBUF)
        wait_dma(i, slot)
        xb = xbufs[slot].astype(jnp.bfloat16)
        p = jnp.dot(xb, wt, preferred_element_type=jnp.float32)
        o_ref[0, pl.ds(i * CH, CH)] = jnp.sqrt(jnp.sum(p * p, axis=1))
        return 0

    lax.fori_loop(0, NCH, step, 0)


def kernel(x, routers, expert_idx):
    w = lax.dynamic_index_in_dim(routers, expert_idx, axis=0,
                                 keepdims=False)
    x2 = x.reshape(ROWS, D)
    out = pl.pallas_call(
        _body,
        in_specs=[
            pl.BlockSpec(memory_space=pl.ANY),
            pl.BlockSpec(memory_space=pltpu.VMEM),
        ],
        out_specs=pl.BlockSpec(memory_space=pltpu.VMEM),
        out_shape=jax.ShapeDtypeStruct((1, ROWS), jnp.float32),
        scratch_shapes=[
            pltpu.VMEM((NBUF, CH, D), jnp.float32),
            pltpu.SemaphoreType.DMA((NBUF,)),
        ],
    )(x2, w.T)
    return out.reshape(4, 4096)
